# baseline (speedup 1.0000x reference)
import functools

import jax
import jax.numpy as jnp
from jax import lax
from jax.experimental import pallas as pl
from jax.experimental.pallas import tpu as pltpu
from jax.experimental.pallas import tpu_sc as plsc

VOCAB = 1000000
EMBED = 64
HIDDEN = 128
NCLS = 50
B = 4096
L = 200

NC = 2
NS = 16
NW = NC * NS
ROWS_PER_W = B // NW
NCHUNK = 2
CHUNK = L // NCHUNK
NBUF = 4
VPR = EMBED // 16


def _sc_pool_body(x_hbm, table_hbm, out_hbm, idx_v, rows_v, out_v, *sems):
    wid = lax.axis_index("s") * NC + lax.axis_index("c")
    pltpu.sync_copy(x_hbm.at[wid], idx_v)

    def issue(r, b):
        for c in range(NCHUNK):
            pltpu.async_copy(
                table_hbm.at[idx_v.at[r, c]],
                rows_v.at[b, pl.ds(c * CHUNK, CHUNK)],
                sems[b],
            )

    def wait(b):
        for c in range(NCHUNK):
            pltpu.make_async_copy(
                table_hbm.at[idx_v.at[0, c]],
                rows_v.at[b, pl.ds(c * CHUNK, CHUNK)],
                sems[b],
            ).wait()

    for b in range(NBUF):
        issue(b, b)

    zero = jnp.zeros((16,), jnp.float32)

    def outer(i, _):
        rr = i * NBUF
        for b in range(NBUF):
            r = rr + b
            wait(b)

            def tok(t, acc):
                base = t * 8
                acc = list(acc)
                for k in range(8):
                    g = (k & 1) * VPR
                    for j in range(VPR):
                        acc[g + j] = acc[g + j] + rows_v[b, base + k, pl.ds(j * 16, 16)]
                return tuple(acc)

            acc = lax.fori_loop(0, L // 8, tok, (zero,) * (2 * VPR))
            for j in range(VPR):
                out_v[r, pl.ds(j * 16, 16)] = acc[j] + acc[VPR + j]

            @pl.when(r + NBUF < ROWS_PER_W)
            def _():
                issue(r + NBUF, b)

        return _

    lax.fori_loop(0, ROWS_PER_W // NBUF, outer, None)
    pltpu.sync_copy(out_v, out_hbm.at[wid])


def _sc_pool(x_blocks, table):
    mesh = plsc.VectorSubcoreMesh(core_axis_name="c", subcore_axis_name="s")
    f = functools.partial(
        pl.kernel,
        out_type=jax.ShapeDtypeStruct((NW, ROWS_PER_W, EMBED), jnp.float32),
        mesh=mesh,
        scratch_types=[
            pltpu.VMEM((ROWS_PER_W, NCHUNK, CHUNK), jnp.int32),
            pltpu.VMEM((NBUF, L, EMBED), jnp.float32),
            pltpu.VMEM((ROWS_PER_W, EMBED), jnp.float32),
        ] + [pltpu.SemaphoreType.DMA] * NBUF,
        compiler_params=pltpu.CompilerParams(use_tc_tiling_on_sc=False),
    )(_sc_pool_body)
    return f(x_blocks, table)


def _mlp_body(e_ref, inv_ref, wh_ref, bh_ref, wf_ref, bf_ref, o_ref):
    e = e_ref[...] * inv_ref[...]
    e = jnp.where(e > 0, e, jnp.exp(e) - 1.0)
    h = lax.dot_general(e, wh_ref[...], (((1,), (1,)), ((), ())),
                        preferred_element_type=jnp.float32) + bh_ref[...]
    h = jnp.where(h > 0, h, jnp.exp(h) - 1.0)
    o = lax.dot_general(h, wf_ref[...], (((1,), (1,)), ((), ())),
                        preferred_element_type=jnp.float32) + bf_ref[...]
    m = jnp.max(o, axis=1, keepdims=True)
    o = o - m
    s = jnp.log(jnp.sum(jnp.exp(o), axis=1, keepdims=True))
    o_ref[...] = o - s


def _tc_mlp(pooled, inv_len, W_h, b_h, W_f, b_f):
    return pl.pallas_call(
        _mlp_body,
        out_shape=jax.ShapeDtypeStruct((B, NCLS), jnp.float32),
    )(pooled, inv_len, W_h, b_h, W_f, b_f)


def kernel(x, x_len, table, W_h, b_h, W_f, b_f):
    x_blocks = x.reshape(NW, ROWS_PER_W, NCHUNK, CHUNK)
    pooled = _sc_pool(x_blocks, table).reshape(B, EMBED)
    inv_len = (1.0 / x_len.astype(jnp.float32)).reshape(B, 1)
    return _tc_mlp(pooled, inv_len, W_h, b_h.reshape(1, HIDDEN),
                   W_f, b_f.reshape(1, NCLS))
